# baseline (speedup 1.0000x reference)
import functools

import jax
import jax.numpy as jnp
from jax import lax
from jax.experimental import pallas as pl
from jax.experimental.pallas import tpu as pltpu
from jax.experimental.pallas import tpu_sc as plsc

B = 16384
C = 20
D = 64
L = 16
NC = 2
NS = 16
NW = NC * NS
EPW = B // NW
GROUP = 16
NG = EPW // GROUP
NGV = 2 * NG
RPG = GROUP * C
IDX_MINOR = 64
IDX_PER_GROUP = RPG // IDX_MINOR
IDX_PER_WORKER = EPW * C // IDX_MINOR


def _rotate(v, idx):
    return lax.gather(
        v, idx[:, None],
        lax.GatherDimensionNumbers(
            offset_dims=(), collapsed_slice_dims=(0,), start_index_map=(0,)),
        slice_sizes=(1,),
        mode=lax.GatherScatterMode.PROMISE_IN_BOUNDS)


@functools.partial(
    pl.kernel,
    mesh=plsc.VectorSubcoreMesh(core_axis_name="c", subcore_axis_name="s"),
    compiler_params=pltpu.CompilerParams(use_tc_tiling_on_sc=False),
    out_type=[
        jax.ShapeDtypeStruct((B,), jnp.float32),
        jax.ShapeDtypeStruct((B,), jnp.float32),
    ],
    scratch_types=[
        pltpu.VMEM((2 * IDX_PER_WORKER // 2, 128), jnp.int32),
        pltpu.VMEM((NGV * GROUP // 128, 128), jnp.int32),
        pltpu.VMEM((RPG, D), jnp.float32),
        pltpu.VMEM((RPG, D), jnp.float32),
        pltpu.VMEM((GROUP, D), jnp.float32),
        pltpu.VMEM((GROUP, D), jnp.float32),
        pltpu.VMEM((NGV * GROUP,), jnp.float32),
        pltpu.SemaphoreType.DMA,
        pltpu.SemaphoreType.DMA,
    ],
)
def _sc_scores(u_tab, w_tab, pos_u_idx, pos_w_idx, neg_u_idx, neg_w_idx,
               out_pos, out_neg, uidx_v, widx_v, rows_v0, rows_v1,
               wrows_v0, wrows_v1, sc_v, sem0, sem1):
    rows_b = (rows_v0, rows_v1)
    wrows_b = (wrows_v0, wrows_v1)
    wid = lax.axis_index("s") * NC + lax.axis_index("c")
    lane = lax.iota(jnp.int32, L)
    rot = {s: lax.bitwise_and(lane + s, L - 1) for s in (8, 4, 2, 1)}
    inv_c = jnp.float32(1.0 / C)
    sems = (sem0, sem1)

    half = IDX_PER_WORKER // 2
    pltpu.sync_copy(
        pos_u_idx.at[pl.ds(wid * half, half)],
        uidx_v.at[pl.ds(0, half)])
    pltpu.sync_copy(
        neg_u_idx.at[pl.ds(wid * half, half)],
        uidx_v.at[pl.ds(half, half)])
    pltpu.sync_copy(pos_w_idx.at[pl.ds(wid * (NG * GROUP // 128), NG * GROUP // 128)],
                    widx_v.at[pl.ds(0, NG * GROUP // 128)])
    pltpu.sync_copy(neg_w_idx.at[pl.ds(wid * (NG * GROUP // 128), NG * GROUP // 128)],
                    widx_v.at[pl.ds(NG * GROUP // 128, NG * GROUP // 128)])

    def issue(v, par):
        base = v * IDX_PER_GROUP
        for k in range(IDX_PER_GROUP):
            c = base + k
            pltpu.async_copy(
                u_tab.at[uidx_v.at[lax.shift_right_logical(c, 1),
                                   pl.ds(lax.bitwise_and(c, 1) * IDX_MINOR,
                                         IDX_MINOR)]],
                rows_b[par].at[pl.ds(k * IDX_MINOR, IDX_MINOR)],
                sems[par])
        pltpu.async_copy(
            w_tab.at[widx_v.at[lax.shift_right_logical(v, 3),
                               pl.ds(lax.bitwise_and(v, 7) * GROUP, GROUP)]],
            wrows_b[par], sems[par])

    def wait(v, par):
        base = v * IDX_PER_GROUP
        for k in range(IDX_PER_GROUP):
            c = base + k
            pltpu.make_async_copy(
                u_tab.at[uidx_v.at[lax.shift_right_logical(c, 1),
                                   pl.ds(lax.bitwise_and(c, 1) * IDX_MINOR,
                                         IDX_MINOR)]],
                rows_b[par].at[pl.ds(k * IDX_MINOR, IDX_MINOR)],
                sems[par]).wait()
        pltpu.make_async_copy(
            w_tab.at[widx_v.at[lax.shift_right_logical(v, 3),
                               pl.ds(lax.bitwise_and(v, 7) * GROUP, GROUP)]],
            wrows_b[par], sems[par]).wait()

    def compute(v, par):
        rows = rows_b[par]
        wrows = wrows_b[par]

        def e_body(e, score):
            base = e * C
            partial = None
            for j in range(D // L):
                a = rows[base, pl.ds(j * L, L)]
                for cc in range(1, C):
                    a = a + rows[base + cc, pl.ds(j * L, L)]
                t = a * wrows[e, pl.ds(j * L, L)]
                partial = t if partial is None else partial + t
            for s in (8, 4, 2, 1):
                partial = partial + _rotate(partial, rot[s])
            return jnp.where(lane == e, partial, score)

        score = lax.fori_loop(0, GROUP, e_body, jnp.zeros((L,), jnp.float32),
                              unroll=2)
        sc_v[pl.ds(pl.multiple_of(v * GROUP, GROUP), GROUP)] = score * inv_c

    issue(jnp.int32(0), 0)
    issue(jnp.int32(1), 1)

    def pair_body(p, carry):
        v0 = 2 * p
        v1 = 2 * p + 1
        wait(v0, 0)
        compute(v0, 0)
        issue(lax.bitwise_and(v0 + 2, NGV - 1), 0)
        wait(v1, 1)
        compute(v1, 1)
        issue(lax.bitwise_and(v1 + 2, NGV - 1), 1)
        return carry

    lax.fori_loop(0, NGV // 2, pair_body, 0)
    wait(jnp.int32(0), 0)
    wait(jnp.int32(1), 1)

    pltpu.sync_copy(sc_v.at[pl.ds(0, EPW)],
                    out_pos.at[pl.ds(wid * EPW, EPW)])
    pltpu.sync_copy(sc_v.at[pl.ds(EPW, EPW)],
                    out_neg.at[pl.ds(wid * EPW, EPW)])


def _tc_loss_body(p_ref, n_ref, o_ref):
    p = p_ref[...]
    n = n_ref[...]
    def logsig(x):
        return jnp.minimum(x, 0.0) - jnp.log1p(jnp.exp(-jnp.abs(x)))
    loss = jnp.sum(logsig(-p)) + jnp.sum(logsig(n))
    o_ref[...] = jnp.broadcast_to(-loss / 128.0, (8, 128))


_tc_loss = pl.pallas_call(
    _tc_loss_body,
    out_shape=jax.ShapeDtypeStruct((8, 128), jnp.float32),
)


def kernel(pos_u, pos_w, neg_u, neg_w, u_table, w_table):
    pos_u_r = pos_u.reshape(B * C // 128, 128)
    neg_u_r = neg_u.reshape(B * C // 128, 128)
    pos_w_r = pos_w.reshape(B // 128, 128)
    neg_w_r = neg_w.reshape(B // 128, 128)
    pos_s, neg_s = _sc_scores(u_table, w_table, pos_u_r, pos_w_r,
                              neg_u_r, neg_w_r)
    out = _tc_loss(pos_s.reshape(128, 128), neg_s.reshape(128, 128))
    return out[0, 0]


# speedup vs baseline: 1.0172x; 1.0172x over previous
import functools

import jax
import jax.numpy as jnp
from jax import lax
from jax.experimental import pallas as pl
from jax.experimental.pallas import tpu as pltpu
from jax.experimental.pallas import tpu_sc as plsc

B = 16384
C = 20
D = 64
L = 16
NC = 2
NS = 16
NW = NC * NS
EPW = B // NW
GROUP = 16
NG = EPW // GROUP
NGV = 2 * NG
RPG = GROUP * C
IDX_MINOR = 64
IDX_PER_GROUP = RPG // IDX_MINOR
IDX_PER_WORKER = EPW * C // IDX_MINOR


def _rotate(v, idx):
    return lax.gather(
        v, idx[:, None],
        lax.GatherDimensionNumbers(
            offset_dims=(), collapsed_slice_dims=(0,), start_index_map=(0,)),
        slice_sizes=(1,),
        mode=lax.GatherScatterMode.PROMISE_IN_BOUNDS)


@functools.partial(
    pl.kernel,
    mesh=plsc.VectorSubcoreMesh(core_axis_name="c", subcore_axis_name="s"),
    compiler_params=pltpu.CompilerParams(use_tc_tiling_on_sc=False),
    out_type=[
        jax.ShapeDtypeStruct((B,), jnp.float32),
        jax.ShapeDtypeStruct((B,), jnp.float32),
    ],
    scratch_types=[
        pltpu.VMEM((IDX_PER_WORKER + NGV * GROUP // 128, 128),
                   jnp.int32),
        pltpu.VMEM((2, RPG, D), jnp.float32),
        pltpu.VMEM((2, GROUP, D), jnp.float32),
        pltpu.VMEM((NGV * GROUP,), jnp.float32),
        pltpu.SemaphoreType.DMA,
        pltpu.SemaphoreType.DMA,
    ],
)
def _sc_scores(u_tab, w_tab, pos_u_idx, pos_w_idx, neg_u_idx, neg_w_idx,
               out_pos, out_neg, idx_v, rows_v, wrows_v, sc_v, sem0, sem1):
    uidx_v = idx_v
    rows_b = (rows_v.at[0], rows_v.at[1])
    wrows_b = (wrows_v.at[0], wrows_v.at[1])
    WOFF = IDX_PER_WORKER
    wid = lax.axis_index("s") * NC + lax.axis_index("c")
    lane = lax.iota(jnp.int32, L)
    rot = {s: lax.bitwise_and(lane + s, L - 1) for s in (8, 4, 2, 1)}
    inv_c = jnp.float32(1.0 / C)
    sems = (sem0, sem1)

    half = IDX_PER_WORKER // 2
    pltpu.sync_copy(
        pos_u_idx.at[pl.ds(wid * half, half)],
        uidx_v.at[pl.ds(0, half)])
    pltpu.sync_copy(
        neg_u_idx.at[pl.ds(wid * half, half)],
        uidx_v.at[pl.ds(half, half)])
    pltpu.sync_copy(pos_w_idx.at[pl.ds(wid * (NG * GROUP // 128), NG * GROUP // 128)],
                    idx_v.at[pl.ds(WOFF, NG * GROUP // 128)])
    pltpu.sync_copy(neg_w_idx.at[pl.ds(wid * (NG * GROUP // 128), NG * GROUP // 128)],
                    idx_v.at[pl.ds(WOFF + NG * GROUP // 128, NG * GROUP // 128)])

    def issue(v, par):
        base = v * IDX_PER_GROUP
        for k in range(IDX_PER_GROUP):
            c = base + k
            pltpu.async_copy(
                u_tab.at[uidx_v.at[lax.shift_right_logical(c, 1),
                                   pl.ds(lax.bitwise_and(c, 1) * IDX_MINOR,
                                         IDX_MINOR)]],
                rows_b[par].at[pl.ds(k * IDX_MINOR, IDX_MINOR)],
                sems[par])
        pltpu.async_copy(
            w_tab.at[idx_v.at[WOFF + lax.shift_right_logical(v, 3),
                              pl.ds(lax.bitwise_and(v, 7) * GROUP, GROUP)]],
            wrows_b[par], sems[par])

    def wait(v, par):
        base = v * IDX_PER_GROUP
        for k in range(IDX_PER_GROUP):
            c = base + k
            pltpu.make_async_copy(
                u_tab.at[uidx_v.at[lax.shift_right_logical(c, 1),
                                   pl.ds(lax.bitwise_and(c, 1) * IDX_MINOR,
                                         IDX_MINOR)]],
                rows_b[par].at[pl.ds(k * IDX_MINOR, IDX_MINOR)],
                sems[par]).wait()
        pltpu.make_async_copy(
            w_tab.at[idx_v.at[WOFF + lax.shift_right_logical(v, 3),
                              pl.ds(lax.bitwise_and(v, 7) * GROUP, GROUP)]],
            wrows_b[par], sems[par]).wait()

    def compute(v, par):
        rows = rows_b[par]
        wrows = wrows_b[par]

        def e_body(e, score):
            base = e * C
            partial = None
            for j in range(D // L):
                a = rows[base, pl.ds(j * L, L)]
                for cc in range(1, C):
                    a = a + rows[base + cc, pl.ds(j * L, L)]
                t = a * wrows[e, pl.ds(j * L, L)]
                partial = t if partial is None else partial + t
            for s in (8, 4, 2, 1):
                partial = partial + _rotate(partial, rot[s])
            return jnp.where(lane == e, partial, score)

        score = lax.fori_loop(0, GROUP, e_body, jnp.zeros((L,), jnp.float32))
        sc_v[pl.ds(pl.multiple_of(v * GROUP, GROUP), GROUP)] = score * inv_c

    issue(jnp.int32(0), 0)
    issue(jnp.int32(1), 1)

    def pair_body(p, carry):
        v0 = 2 * p
        v1 = 2 * p + 1
        wait(v0, 0)
        compute(v0, 0)
        issue(lax.bitwise_and(v0 + 2, NGV - 1), 0)
        wait(v1, 1)
        compute(v1, 1)
        issue(lax.bitwise_and(v1 + 2, NGV - 1), 1)
        return carry

    lax.fori_loop(0, NGV // 2, pair_body, 0)
    wait(jnp.int32(0), 0)
    wait(jnp.int32(1), 1)

    pltpu.sync_copy(sc_v.at[pl.ds(0, EPW)],
                    out_pos.at[pl.ds(wid * EPW, EPW)])
    pltpu.sync_copy(sc_v.at[pl.ds(EPW, EPW)],
                    out_neg.at[pl.ds(wid * EPW, EPW)])


def _tc_loss_body(p_ref, n_ref, o_ref):
    p = p_ref[...]
    n = n_ref[...]
    def logsig(x):
        return jnp.minimum(x, 0.0) - jnp.log1p(jnp.exp(-jnp.abs(x)))
    loss = jnp.sum(logsig(-p)) + jnp.sum(logsig(n))
    o_ref[...] = jnp.broadcast_to(-loss / 128.0, (8, 128))


_tc_loss = pl.pallas_call(
    _tc_loss_body,
    out_shape=jax.ShapeDtypeStruct((8, 128), jnp.float32),
)


def kernel(pos_u, pos_w, neg_u, neg_w, u_table, w_table):
    pos_u_r = pos_u.reshape(B * C // 128, 128)
    neg_u_r = neg_u.reshape(B * C // 128, 128)
    pos_w_r = pos_w.reshape(B // 128, 128)
    neg_w_r = neg_w.reshape(B // 128, 128)
    pos_s, neg_s = _sc_scores(u_table, w_table, pos_u_r, pos_w_r,
                              neg_u_r, neg_w_r)
    out = _tc_loss(pos_s.reshape(128, 128), neg_s.reshape(128, 128))
    return out[0, 0]
